# baseline (speedup 1.0000x reference)
import functools

import jax
import jax.numpy as jnp
from jax import lax
from jax.experimental import pallas as pl
from jax.experimental.pallas import tpu as pltpu
from jax.experimental.pallas import tpu_sc as plsc

B, L, D = 1024, 200, 128
R = B * L
NW = 32
RPW = R // NW
C = 128
NCHUNK = RPW // C
NBUF = 3
DEPTH = 2
NOUTER = -(-NCHUNK // NBUF)
LANES = 16
NSUB = C // LANES

_mesh = plsc.VectorSubcoreMesh(core_axis_name="c", subcore_axis_name="s")


@functools.partial(
    pl.kernel,
    mesh=_mesh,
    out_type=jax.ShapeDtypeStruct((R * D,), jnp.float32),
    scratch_types=[
        pltpu.VMEM((C * D,), jnp.float32),
        pltpu.VMEM((C * D,), jnp.float32),
        pltpu.VMEM((C * D,), jnp.float32),
        pltpu.VMEM((C, D), jnp.float32),
        pltpu.VMEM((C, D), jnp.float32),
        pltpu.VMEM((C, D), jnp.float32),
        pltpu.VMEM((C + LANES,), jnp.int32),
        pltpu.VMEM((C + LANES,), jnp.int32),
        pltpu.VMEM((C + LANES,), jnp.int32),
        pltpu.VMEM((RPW,), jnp.int32),
        pltpu.SemaphoreType.DMA,
        pltpu.SemaphoreType.DMA,
        pltpu.SemaphoreType.DMA,
        pltpu.SemaphoreType.DMA,
        pltpu.SemaphoreType.DMA,
        pltpu.SemaphoreType.DMA,
    ],
    compiler_params=pltpu.CompilerParams(needs_layout_passes=False),
)
def _masked_copy(x_hbm, seq_hbm, out_hbm,
                 outb0, outb1, outb2, cmp0, cmp1, cmp2, idx0, idx1, idx2,
                 seq_all, isem0, isem1, isem2, osem0, osem1, osem2):
    wid = lax.axis_index("s") * 2 + lax.axis_index("c")
    base = wid * RPW
    outbs = (outb0, outb1, outb2)
    cmps = (cmp0, cmp1, cmp2)
    idxs = (idx0, idx1, idx2)
    isems = (isem0, isem1, isem2)
    osems = (osem0, osem1, osem2)
    lane = lax.iota(jnp.int32, LANES)
    zeros = jnp.zeros((LANES,), jnp.float32)

    def count_keep(cj):
        def cn(g, cnt):
            svec = seq_all[pl.ds(cj * C + g * LANES, LANES)]
            return cnt + plsc.all_reduce_population_count(svec != 0)[0]

        return lax.fori_loop(0, NSUB, cn, jnp.int32(0))

    def build_and_gather(b, cj):
        rb = base + cj * C

        def bg(g, cnt):
            svec = seq_all[pl.ds(cj * C + g * LANES, LANES)]
            keep = svec != 0
            rows = rb + g * LANES + lane
            plsc.store_compressed(idxs[b].at[pl.ds(cnt, LANES)], rows, mask=keep)
            return cnt + plsc.all_reduce_population_count(keep)[0]

        n = lax.fori_loop(0, NSUB, bg, jnp.int32(0))
        idxs[b][pl.ds(n, LANES)] = rb + 0 * lane
        n16 = (n + LANES - 1) // LANES

        def gi(j, c):
            pltpu.async_copy(
                x_hbm.at[idxs[b].at[pl.ds(j * LANES, LANES)]],
                cmps[b].at[pl.ds(j * LANES, LANES)], isems[b])
            return c

        lax.fori_loop(0, n16, gi, 0)

    def wait_gathers(b, n16):
        def wg(j, c):
            pltpu.make_async_copy(
                x_hbm.at[idxs[b].at[pl.ds(0, LANES)]],
                cmps[b].at[pl.ds(0, LANES)], isems[b]).wait()
            return c

        lax.fori_loop(0, n16, wg, 0)

    def expand(b, cj):
        def ex(g, pos):
            svec = seq_all[pl.ds(cj * C + g * LANES, LANES)]
            for k in range(LANES):
                keep_k = svec[k] != 0
                r = g * LANES + k

                @pl.when(keep_k)
                def _():
                    for j in range(D // LANES):
                        outbs[b][pl.ds(r * D + j * LANES, LANES)] = (
                            cmps[b][pos, pl.ds(j * LANES, LANES)])

                @pl.when(jnp.logical_not(keep_k))
                def _():
                    for j in range(D // LANES):
                        outbs[b][pl.ds(r * D + j * LANES, LANES)] = zeros

                pos = pos + keep_k.astype(jnp.int32)
            return pos

        lax.fori_loop(0, NSUB, ex, jnp.int32(0))

    def start_out(b, ci):
        rb = base + ci * C
        pltpu.async_copy(outbs[b], out_hbm.at[pl.ds(rb * D, C * D)], osems[b])

    def wait_out(b, ci):
        rb = base + ci * C
        pltpu.make_async_copy(
            outbs[b], out_hbm.at[pl.ds(rb * D, C * D)], osems[b]).wait()

    pltpu.sync_copy(seq_hbm.at[pl.ds(base, RPW)], seq_all)

    for b in range(DEPTH):
        build_and_gather(b, b)

    def outer_body(o, carry):
        for b in range(NBUF):
            ci = o * NBUF + b

            @pl.when(ci < NCHUNK)
            def _():
                n = count_keep(ci)
                n16 = (n + LANES - 1) // LANES
                wait_gathers(b, n16)
                expand(b, ci)
                start_out(b, ci)

                bn = (b + DEPTH) % NBUF
                cj = ci + DEPTH

                @pl.when(cj < NCHUNK)
                def _():
                    @pl.when(cj >= NBUF)
                    def _():
                        wait_out(bn, cj - NBUF)

                    build_and_gather(bn, cj)
        return carry

    lax.fori_loop(0, NOUTER, outer_body, 0)

    for b in range(NBUF):
        ci_last = NCHUNK - NBUF + b
        wait_out(ci_last % NBUF, ci_last)


def kernel(x, item_seq):
    xf = x.reshape(R, D)
    seq = item_seq.reshape(R).astype(jnp.int32)
    out = _masked_copy(xf, seq)
    return out.reshape(B, L, D)


# speedup vs baseline: 2.6281x; 2.6281x over previous
import functools

import jax
import jax.numpy as jnp
from jax import lax
from jax.experimental import pallas as pl
from jax.experimental.pallas import tpu as pltpu
from jax.experimental.pallas import tpu_sc as plsc

B, L, D = 1024, 200, 128
R = B * L
NW = 32
RPW = R // NW
C = 160
NCHUNK = RPW // C
NBUF = 4
NOUTER = NCHUNK // NBUF
LANES = 16

_mesh = plsc.VectorSubcoreMesh(core_axis_name="c", subcore_axis_name="s")


@functools.partial(
    pl.kernel,
    mesh=_mesh,
    out_type=jax.ShapeDtypeStruct((R * D,), jnp.float32),
    scratch_types=[
        pltpu.VMEM((C * D,), jnp.float32),
        pltpu.VMEM((C * D,), jnp.float32),
        pltpu.VMEM((C * D,), jnp.float32),
        pltpu.VMEM((C * D,), jnp.float32),
        pltpu.VMEM((C,), jnp.int32),
        pltpu.VMEM((C,), jnp.int32),
        pltpu.VMEM((C,), jnp.int32),
        pltpu.VMEM((C,), jnp.int32),
        pltpu.SemaphoreType.DMA,
        pltpu.SemaphoreType.DMA,
        pltpu.SemaphoreType.DMA,
        pltpu.SemaphoreType.DMA,
        pltpu.SemaphoreType.DMA,
        pltpu.SemaphoreType.DMA,
        pltpu.SemaphoreType.DMA,
        pltpu.SemaphoreType.DMA,
    ],
    compiler_params=pltpu.CompilerParams(needs_layout_passes=False),
)
def _masked_copy(x_hbm, seq_hbm, out_hbm,
                 buf0, buf1, buf2, buf3, sq0, sq1, sq2, sq3,
                 isem0, isem1, isem2, isem3, osem0, osem1, osem2, osem3):
    wid = lax.axis_index("s") * 2 + lax.axis_index("c")
    base = wid * RPW
    bufs = (buf0, buf1, buf2, buf3)
    sqs = (sq0, sq1, sq2, sq3)
    isems = (isem0, isem1, isem2, isem3)
    osems = (osem0, osem1, osem2, osem3)
    zeros = jnp.zeros((LANES,), jnp.float32)

    def start_in(b, ci):
        rb = base + ci * C
        pltpu.async_copy(x_hbm.at[pl.ds(rb * D, C * D)], bufs[b], isems[b])
        pltpu.async_copy(seq_hbm.at[pl.ds(rb, C)], sqs[b], isems[b])

    def wait_in(b, ci):
        rb = base + ci * C
        pltpu.make_async_copy(
            x_hbm.at[pl.ds(rb * D, C * D)], bufs[b], isems[b]).wait()
        pltpu.make_async_copy(
            seq_hbm.at[pl.ds(rb, C)], sqs[b], isems[b]).wait()

    def start_out(b, ci):
        rb = base + ci * C
        pltpu.async_copy(bufs[b], out_hbm.at[pl.ds(rb * D, C * D)], osems[b])

    def wait_out(b, ci):
        rb = base + ci * C
        pltpu.make_async_copy(
            bufs[b], out_hbm.at[pl.ds(rb * D, C * D)], osems[b]).wait()

    start_in(0, 0)
    start_in(1, 1)

    def outer_body(o, carry):
        for b in range(NBUF):
            ci = o * NBUF + b
            wait_in(b, ci)

            def grp_body(g, c2):
                svec = sqs[b][pl.ds(g * LANES, LANES)]
                gbase = g * (LANES * D)
                for k in range(LANES):
                    @pl.when(svec[k] == 0)
                    def _():
                        rb2 = gbase + k * D
                        for j in range(D // LANES):
                            bufs[b][pl.ds(rb2 + j * LANES, LANES)] = zeros
                return c2

            lax.fori_loop(0, C // LANES, grp_body, 0)
            start_out(b, ci)

            bn = (b + 2) % NBUF

            @pl.when(ci + 2 < NCHUNK)
            def _():
                @pl.when(ci >= 2)
                def _():
                    wait_out(bn, ci - 2)

                start_in(bn, ci + 2)
        return carry

    lax.fori_loop(0, NOUTER, outer_body, 0)

    for b in range(NBUF):
        wait_out(b, NCHUNK - NBUF + b)


def kernel(x, item_seq):
    xf = x.reshape(R * D)
    seq = item_seq.reshape(R).astype(jnp.int32)
    out = _masked_copy(xf, seq)
    return out.reshape(B, L, D)
